# baseline (speedup 1.0000x reference)
import jax
import jax.numpy as jnp
from jax import lax
from jax.experimental import pallas as pl
from jax.experimental.pallas import tpu as pltpu
from jax.experimental.pallas import tpu_sc as plsc

_B = 4096
_L = 50
_D = 128
_MD = 50
_OUT = _D + _MD
_NW = 32
_BW = _B // _NW
_NG = _BW // 16


def _body(sent_t, mask_t, w_word, tgt_e, wm_pad, out, idx_v, mask_v, tpos_v,
          tv_v, wm_v, gath_a, gath_b, gath_c, blk_a, blk_b,
          sem_ga, sem_gb, sem_gc, sem_oa, sem_ob):
  wid = lax.axis_index("s") * 2 + lax.axis_index("c")
  b0 = wid * _BW
  pltpu.sync_copy(sent_t.at[:, pl.ds(b0, _BW)], idx_v)
  pltpu.sync_copy(mask_t.at[:, pl.ds(b0, _BW)], mask_v)
  pltpu.sync_copy(tgt_e, tv_v)
  pltpu.sync_copy(wm_pad, wm_v)
  gath = (gath_a, gath_b, gath_c)
  blk = (blk_a, blk_b)
  sem_g = (sem_ga, sem_gb, sem_gc)
  sem_o = (sem_oa, sem_ob)

  iota = lax.iota(jnp.int32, 16)
  tvecs = [tv_v[16 * j:16 * j + 16] for j in range(_D // 16)]

  for g in range(_NG):

    def amax(l, tp):
      mv = mask_v[l, 16 * g:16 * g + 16]
      return jnp.where((mv > 0) & (tp >= _L), l, tp)

    tp = lax.fori_loop(0, _L, amax, jnp.full((16,), _L, jnp.int32))
    tpos_v[16 * g:16 * g + 16] = jnp.where(tp >= _L, 0, tp)

  def gather(l, par):
    pltpu.async_copy(w_word.at[idx_v.at[l]], gath[par], sem_g[par])

  def wait_gather(par):
    pltpu.make_async_copy(w_word.at[idx_v.at[0]], gath[par],
                          sem_g[par]).wait()

  def put(l, par):
    pltpu.async_copy(blk[par], out.at[l, :, pl.ds(b0, _BW)], sem_o[par])

  def wait_put(par):
    pltpu.make_async_copy(blk[par], out.at[0, :, pl.ds(b0, _BW)],
                          sem_o[par]).wait()

  def assemble(l, g_par, b_par):
    g_v = gath[g_par]
    b_v = blk[b_par]

    def hit_scan(g, _):
      pred = tpos_v[pl.ds(16 * g, 16)] == l

      def cond(f):
        return f < 16

      def hit_body(f):
        r = 16 * g + f
        for j in range(_D // 16):
          g_v[r, 16 * j:16 * j + 16] = tvecs[j]
        nf = plsc.all_reduce_ffs(pred & (iota > f))[0]
        return nf

      lax.while_loop(cond, hit_body, plsc.all_reduce_ffs(pred)[0])
      return 0

    lax.fori_loop(0, _NG, hit_scan, 0)

    def trans(b, _):
      colv = jnp.full((16,), b, jnp.int32)
      for j in range(_D // 16):
        vals = g_v[b, 16 * j:16 * j + 16]
        plsc.store_scatter(b_v, [iota + 16 * j, colv], vals)
      return 0

    lax.fori_loop(0, _BW, trans, 0, unroll=2)

    sels = [mask_v[l, 16 * g:16 * g + 16] > 0 for g in range(_NG)]

    def tail(t, _):
      w0 = plsc.load_gather(wm_v, [jnp.full((16,), t, jnp.int32)])
      w1 = plsc.load_gather(wm_v, [jnp.full((16,), 64 + t, jnp.int32)])
      outs = [jnp.where(sels[g], w1, w0) for g in range(_NG)]
      for g in range(_NG):
        b_v[_D + t, 16 * g:16 * g + 16] = outs[g]
      return 0

    lax.fori_loop(0, _MD, tail, 0, unroll=2)

  gather(0, 0)
  gather(1, 1)
  for l in range(2):
    wait_gather(l % 3)
    gather(l + 2, (l + 2) % 3)
    assemble(l, l % 3, l % 2)
    put(l, l % 2)

  def step(l6, _):
    for k in range(6):
      l = 2 + 6 * l6 + k
      wait_gather((2 + k) % 3)

      @pl.when(l + 2 < _L)
      def _():
        gather(l + 2, (4 + k) % 3)

      wait_put(k % 2)
      assemble(l, (2 + k) % 3, k % 2)
      put(l, k % 2)
    return 0

  lax.fori_loop(0, (_L - 2) // 6, step, 0)
  wait_put(0)
  wait_put(1)


def _run(sent_t, mask_t, w_word, tgt_e, wm_pad):
  mesh = plsc.VectorSubcoreMesh(core_axis_name="c", subcore_axis_name="s")
  f = pl.kernel(
      _body,
      out_type=jax.ShapeDtypeStruct((_L, _OUT, _B), jnp.float32),
      mesh=mesh,
      compiler_params=pltpu.CompilerParams(needs_layout_passes=False),
      scratch_types=[
          pltpu.VMEM((_L, _BW), jnp.int32),
          pltpu.VMEM((_L, _BW), jnp.int32),
          pltpu.VMEM((_BW,), jnp.int32),
          pltpu.VMEM((_D,), jnp.float32),
          pltpu.VMEM((128,), jnp.float32),
          pltpu.VMEM((_BW, _D), jnp.float32),
          pltpu.VMEM((_BW, _D), jnp.float32),
          pltpu.VMEM((_BW, _D), jnp.float32),
          pltpu.VMEM((_OUT, _BW), jnp.float32),
          pltpu.VMEM((_OUT, _BW), jnp.float32),
          pltpu.SemaphoreType.DMA,
          pltpu.SemaphoreType.DMA,
          pltpu.SemaphoreType.DMA,
          pltpu.SemaphoreType.DMA,
          pltpu.SemaphoreType.DMA,
      ],
  )
  return f(sent_t, mask_t, w_word, tgt_e, wm_pad)


@jax.jit
def _run_all(sent, mask, W_word, target_emb, W_mask):
  sent_t = sent.T
  mask_t = mask.T
  wm_pad = jnp.pad(W_mask, ((0, 0), (0, 64 - _MD))).reshape(-1)
  out = _run(sent_t, mask_t, W_word, target_emb, wm_pad)
  return lax.transpose(out, (2, 0, 1))


def kernel(sent, mask, W_word, target_emb, W_mask):
  return _run_all(sent, mask, W_word, target_emb, W_mask)


# speedup vs baseline: 1.5988x; 1.5988x over previous
import jax
import jax.numpy as jnp
from jax import lax
from jax.experimental import pallas as pl
from jax.experimental.pallas import tpu as pltpu
from jax.experimental.pallas import tpu_sc as plsc

_B = 4096
_L = 50
_D = 128
_MD = 50
_OUT = _D + _MD
_NW = 32
_BPW = _B // _NW
_CB = 2
_CR = _CB * _L
_NCH = _BPW // _CB
_PW = _BPW * _L
_TOFF = (0, 16, 32, 34)


def _body(sent_r, mask_r, w_word, tgt_e, wm_pad, out, idx_v, mask_v, tv_v,
          wm_v, out_a, out_b, out_c, sem_ga, sem_gb, sem_gc,
          sem_oa, sem_ob, sem_oc):
  wid = lax.axis_index("s") * 2 + lax.axis_index("c")
  pltpu.sync_copy(sent_r.at[wid], idx_v)
  pltpu.sync_copy(mask_r.at[wid], mask_v.at[pl.ds(0, _PW)])
  pltpu.sync_copy(tgt_e, tv_v)
  pltpu.sync_copy(wm_pad, wm_v)
  out_v = (out_a, out_b, out_c)
  sem_g = (sem_ga, sem_gb, sem_gc)
  sem_o = (sem_oa, sem_ob, sem_oc)
  bbase = wid * _BPW

  tvecs = [tv_v[16 * j:16 * j + 16] for j in range(_D // 16)]
  w0s = [wm_v[off:off + 16] for off in _TOFF]
  w1s = [wm_v[64 + off:64 + off + 16] for off in _TOFF]

  def assemble(c, par):
    o_v = out_v[par]
    for bl in range(_CB):

      def row(rr, _):
        fi = jnp.full((16,), c * _CR + bl * _L + rr, jnp.int32)
        sel = plsc.load_gather(mask_v, [fi]) > 0
        for k, off in enumerate(_TOFF):
          o_v[bl, rr, _D + off:_D + off + 16] = jnp.where(
              sel, w1s[k], w0s[k])
        return 0

      lax.fori_loop(0, _L, row, 0, unroll=2)

      o = c * _CR + bl * _L
      tpos = jnp.int32(0)
      for j in reversed(range(4)):
        mv = mask_v[pl.ds(o + 16 * j, 16)]
        pos = lax.iota(jnp.int32, 16) + 16 * j
        ok = (mv > 0) & (pos < _L)
        f = plsc.all_reduce_ffs(ok)[0]
        tpos = jnp.where(f < 16, 16 * j + f, tpos)
      for j in range(_D // 16):
        o_v[bl, tpos, 16 * j:16 * j + 16] = tvecs[j]

  def gather(c, par):
    for bl in range(_CB):
      pltpu.async_copy(w_word.at[idx_v.at[c, bl]],
                       out_v[par].at[bl, :, pl.ds(0, _D)], sem_g[par])

  def wait_gather(par):
    for bl in range(_CB):
      pltpu.make_async_copy(w_word.at[idx_v.at[0, bl]],
                            out_v[par].at[bl, :, pl.ds(0, _D)],
                            sem_g[par]).wait()

  def put(c, par):
    pltpu.async_copy(out_v[par], out.at[pl.ds(bbase + c * _CB, _CB)],
                     sem_o[par])

  def wait_put(par):
    pltpu.make_async_copy(out_v[par], out.at[pl.ds(bbase, _CB)],
                          sem_o[par]).wait()

  gather(0, 0)
  gather(1, 1)
  wait_gather(0)
  assemble(0, 0)
  put(0, 0)
  gather(2, 2)

  def step(c3, _):
    for k in range(3):
      c = 1 + 3 * c3 + k
      par = (1 + k) % 3
      wait_gather(par)
      assemble(c, par)
      put(c, par)
      wait_put(k % 3)

      @pl.when(c + 2 < _NCH)
      def _():
        gather(c + 2, k % 3)

    return 0

  lax.fori_loop(0, (_NCH - 1) // 3, step, 0)
  wait_put(0)


def _run(sent_r, mask_r, w_word, tgt_e, wm_pad):
  mesh = plsc.VectorSubcoreMesh(core_axis_name="c", subcore_axis_name="s")
  f = pl.kernel(
      _body,
      out_type=jax.ShapeDtypeStruct((_B, _L, _OUT), jnp.float32),
      mesh=mesh,
      compiler_params=pltpu.CompilerParams(needs_layout_passes=False),
      scratch_types=[
          pltpu.VMEM((_NCH, _CB, _L), jnp.int32),
          pltpu.VMEM((_PW + 16,), jnp.int32),
          pltpu.VMEM((_D,), jnp.float32),
          pltpu.VMEM((128,), jnp.float32),
          pltpu.VMEM((_CB, _L, _OUT), jnp.float32),
          pltpu.VMEM((_CB, _L, _OUT), jnp.float32),
          pltpu.VMEM((_CB, _L, _OUT), jnp.float32),
          pltpu.SemaphoreType.DMA,
          pltpu.SemaphoreType.DMA,
          pltpu.SemaphoreType.DMA,
          pltpu.SemaphoreType.DMA,
          pltpu.SemaphoreType.DMA,
          pltpu.SemaphoreType.DMA,
      ],
  )
  return f(sent_r, mask_r, w_word, tgt_e, wm_pad)


@jax.jit
def _run_all(sent, mask, W_word, target_emb, W_mask):
  sent_r = sent.reshape(_NW, _NCH, _CB, _L)
  mask_r = mask.reshape(_NW, _PW)
  wm_pad = jnp.pad(W_mask, ((0, 0), (0, 64 - _MD))).reshape(-1)
  return _run(sent_r, mask_r, W_word, target_emb, wm_pad)


def kernel(sent, mask, W_word, target_emb, W_mask):
  return _run_all(sent, mask, W_word, target_emb, W_mask)
